# baseline (speedup 1.0000x reference)
import functools

import jax
import jax.numpy as jnp
from jax import lax
from jax.experimental import pallas as pl
from jax.experimental.pallas import tpu as pltpu
from jax.experimental.pallas import tpu_sc as plsc

N_CLS = 4096
N_CTX = 16
MAX_NAME = 16
D = 768
MAX_LEN = 1 + N_CTX + MAX_NAME + 1
HEAD = 1 + N_CTX
TAIL = MAX_NAME + 1
NLANE = D // 16

NC = 2
NS = 16
NW = NC * NS
PER_TILE = N_CLS // NW
CHUNK = 2
NBUF = 2
STEPS = PER_TILE // (CHUNK * NBUF)


def _body(table_hbm, ctx_hbm, ct_hbm, lens_hbm, par_hbm,
          out_hbm, mask_hbm,
          buf, ct_v, lens_v, mask_v, par_v,
          idx00, idx01, idx10, idx11, idx_cls,
          gsem0, gsem1, osem0, osem1):
    idx_refs = ((idx00, idx01), (idx10, idx11))
    gsems = (gsem0, gsem1)
    osems = (osem0, osem1)

    wid = lax.axis_index("s") * NC + lax.axis_index("c")
    base = wid * PER_TILE
    iota = lax.broadcasted_iota(jnp.int32, (16,), 0)
    zero16 = jnp.zeros((16,), jnp.float32)

    pltpu.sync_copy(par_hbm, par_v)
    pltpu.sync_copy(ct_hbm.at[pl.ds(base, PER_TILE)], ct_v)
    pltpu.sync_copy(lens_hbm.at[pl.ds(base, PER_TILE)], lens_v)
    cls_v = plsc.load_gather(par_v, [iota * 0])
    sep_v = plsc.load_gather(par_v, [iota * 0 + 1])
    plsc.store_scatter(idx_cls, [iota], cls_v, mask=iota < 8)
    for b in range(NBUF):
        for c in range(CHUNK):
            pltpu.async_copy(table_hbm.at[idx_cls],
                             buf.at[b, c, pl.ds(0, 8)], gsems[b]).wait()
            pltpu.sync_copy(ctx_hbm, buf.at[b, c, pl.ds(1, N_CTX)])

    def step(s, carry):
        lens_pair = [[None] * CHUNK for _ in range(NBUF)]
        for b in range(NBUF):
            g = s * NBUF + b

            @pl.when(g >= NBUF)
            def _():
                pltpu.make_async_copy(buf.at[b], out_hbm.at[pl.ds(0, CHUNK)],
                                      osems[b]).wait()

            for c in range(CHUNK):
                local = g * CHUNK + c
                lsp = jnp.full((16,), local, jnp.int32)
                tok = plsc.load_gather(ct_v, [lsp, iota])
                lenv = plsc.load_gather(lens_v, [lsp])
                lens_pair[b][c] = lenv
                idx16 = jnp.where(iota < lenv, tok, sep_v)
                iref = idx_refs[b][c]
                iref[pl.ds(0, 16)] = idx16
                plsc.store_scatter(iref, [iota * 0 + MAX_NAME], sep_v,
                                   mask=iota == 0)

                cur = lenv + 18
                plsc.store_scatter(mask_v, [lsp, iota],
                                   jnp.full((16,), 1, jnp.int32))
                plsc.store_scatter(mask_v, [lsp, iota + 16],
                                   (iota + 16 < cur).astype(jnp.int32))
                plsc.store_scatter(mask_v, [lsp, iota + 32],
                                   (iota + 32 < cur).astype(jnp.int32),
                                   mask=iota < 2)

                pltpu.async_copy(
                    table_hbm.at[iref], buf.at[b, c, pl.ds(HEAD, TAIL)],
                    gsems[b])

        for b in range(NBUF):
            g = s * NBUF + b
            c0 = base + g * CHUNK
            for c in range(CHUNK):
                pltpu.make_async_copy(
                    table_hbm.at[idx_refs[b][c]],
                    buf.at[b, c, pl.ds(HEAD, TAIL)], gsems[b]).wait()
            for c in range(CHUNK):
                lenv = lens_pair[b][c]
                for srow in range(2, TAIL):
                    keep = srow <= lenv
                    for k in range(NLANE):
                        sl = pl.ds(k * 16, 16)
                        buf[b, c, HEAD + srow, sl] = jnp.where(
                            keep, buf[b, c, HEAD + srow, sl], zero16)
            pltpu.async_copy(buf.at[b], out_hbm.at[pl.ds(c0, CHUNK)], osems[b])
        return carry

    lax.fori_loop(0, STEPS, step, 0)

    for b in range(NBUF):
        pltpu.make_async_copy(buf.at[b], out_hbm.at[pl.ds(0, CHUNK)],
                              osems[b]).wait()
    pltpu.sync_copy(mask_v, mask_hbm.at[pl.ds(base, PER_TILE)])


def _sc_call(table, ctx, class_tokens, lens, par):
    mesh = plsc.VectorSubcoreMesh(core_axis_name="c", subcore_axis_name="s")
    f = pl.kernel(
        _body,
        mesh=mesh,
        compiler_params=pltpu.CompilerParams(use_tc_tiling_on_sc=False,
                                             needs_layout_passes=False),
        out_type=(
            jax.ShapeDtypeStruct((N_CLS, MAX_LEN, D), jnp.float32),
            jax.ShapeDtypeStruct((N_CLS, MAX_LEN), jnp.int32),
        ),
        scratch_types=[
            pltpu.VMEM((NBUF, CHUNK, MAX_LEN, D), jnp.float32),
            pltpu.VMEM((PER_TILE, MAX_NAME), jnp.int32),
            pltpu.VMEM((PER_TILE,), jnp.int32),
            pltpu.VMEM((PER_TILE, MAX_LEN), jnp.int32),
            pltpu.VMEM((8,), jnp.int32),
            pltpu.VMEM((TAIL,), jnp.int32),
            pltpu.VMEM((TAIL,), jnp.int32),
            pltpu.VMEM((TAIL,), jnp.int32),
            pltpu.VMEM((TAIL,), jnp.int32),
            pltpu.VMEM((8,), jnp.int32),
            pltpu.SemaphoreType.DMA,
            pltpu.SemaphoreType.DMA,
            pltpu.SemaphoreType.DMA,
            pltpu.SemaphoreType.DMA,
        ],
    )
    return f(table, ctx, class_tokens, lens, par)


def kernel(table, ctx, class_tokens, lens, cls_id, sep_id):
    par = (jnp.zeros((8,), jnp.int32)
           .at[0].set(jnp.asarray(cls_id, jnp.int32))
           .at[1].set(jnp.asarray(sep_id, jnp.int32)))
    out_embeds, out_mask = _sc_call(table, ctx, class_tokens, lens, par)
    return out_embeds, out_mask


# speedup vs baseline: 1.1033x; 1.1033x over previous
import functools

import jax
import jax.numpy as jnp
from jax import lax
from jax.experimental import pallas as pl
from jax.experimental.pallas import tpu as pltpu
from jax.experimental.pallas import tpu_sc as plsc

N_CLS = 4096
N_CTX = 16
MAX_NAME = 16
D = 768
MAX_LEN = 1 + N_CTX + MAX_NAME + 1
HEAD = 1 + N_CTX
TAIL = MAX_NAME + 1

NC = 2
NS = 16
NW = NC * NS
PER_TILE = N_CLS // NW
K = 2
NBUF = 4
STEPS = PER_TILE // (K * NBUF)


def _sc_body(table_hbm, ct_hbm, lens_hbm, par_hbm,
             t_hbm,
             stag0, stag1, stag2, stag3,
             gidx0, gidx1, gidx2, gidx3,
             ct_v, lens_v, par_v,
             gsem0, gsem1, gsem2, gsem3,
             osem0, osem1, osem2, osem3):
    stags = (stag0, stag1, stag2, stag3)
    gidxs = (gidx0, gidx1, gidx2, gidx3)
    gsems = (gsem0, gsem1, gsem2, gsem3)
    osems = (osem0, osem1, osem2, osem3)

    wid = lax.axis_index("s") * NC + lax.axis_index("c")
    base = wid * PER_TILE
    iota = lax.broadcasted_iota(jnp.int32, (16,), 0)

    pltpu.sync_copy(par_hbm, par_v)
    pltpu.sync_copy(ct_hbm.at[pl.ds(base, PER_TILE)], ct_v)
    pltpu.sync_copy(lens_hbm.at[pl.ds(base, PER_TILE)], lens_v)
    sep_v = plsc.load_gather(par_v, [iota * 0 + 1])

    def fill_idx(b, g):
        for c in range(K):
            local = g * K + c
            lsp = jnp.full((16,), local, jnp.int32)
            tok = plsc.load_gather(ct_v, [lsp, iota])
            lenv = plsc.load_gather(lens_v, [lsp])
            idx16 = jnp.where(iota < lenv, tok, sep_v)
            plsc.store_scatter(gidxs[b], [iota * 0 + (c * TAIL) + iota], idx16)
            plsc.store_scatter(gidxs[b], [iota * 0 + (c * TAIL + 16)], sep_v,
                               mask=iota == 0)

    def step(s, carry):
        for b in range(NBUF):
            g = s * NBUF + b

            @pl.when(g >= NBUF)
            def _():
                pltpu.make_async_copy(
                    stags[b], t_hbm.at[pl.ds(0, K * TAIL)], osems[b]).wait()

            fill_idx(b, g)
            pltpu.async_copy(table_hbm.at[gidxs[b]], stags[b], gsems[b])
        for b in range(NBUF):
            g = s * NBUF + b
            r0 = (base + g * K) * TAIL
            pltpu.make_async_copy(
                table_hbm.at[gidxs[b]], stags[b], gsems[b]).wait()
            pltpu.async_copy(stags[b], t_hbm.at[pl.ds(r0, K * TAIL)], osems[b])
        return carry

    lax.fori_loop(0, STEPS, step, 0)
    for b in range(NBUF):
        pltpu.make_async_copy(stags[b], t_hbm.at[pl.ds(0, K * TAIL)],
                              osems[b]).wait()


def _sc_gather(table, class_tokens, lens, par):
    mesh = plsc.VectorSubcoreMesh(core_axis_name="c", subcore_axis_name="s")
    f = pl.kernel(
        _sc_body,
        mesh=mesh,
        compiler_params=pltpu.CompilerParams(use_tc_tiling_on_sc=False,
                                             needs_layout_passes=False),
        out_type=jax.ShapeDtypeStruct((N_CLS * TAIL, D), jnp.float32),
        scratch_types=(
            [pltpu.VMEM((K * TAIL, D), jnp.float32)] * NBUF
            + [pltpu.VMEM((K * TAIL,), jnp.int32)] * NBUF
            + [
                pltpu.VMEM((PER_TILE, MAX_NAME), jnp.int32),
                pltpu.VMEM((PER_TILE,), jnp.int32),
                pltpu.VMEM((8,), jnp.int32),
            ]
            + [pltpu.SemaphoreType.DMA] * (2 * NBUF)
        ),
    )
    return f(table, class_tokens, lens, par)


BC = 64


def _tc_body(t_ref, base_ref, lens_ref, out_ref, mask_ref):
    lenb = lens_ref[...]
    s_iota = lax.broadcasted_iota(jnp.int32, (BC, TAIL, 1), 1)
    tail = jnp.where(s_iota <= lenb[:, :, None], t_ref[...], 0.0)
    head = jnp.broadcast_to(base_ref[...][None], (BC, HEAD, D))
    out_ref[...] = jnp.concatenate([head, tail], axis=1)
    p_iota = lax.broadcasted_iota(jnp.int32, (BC, MAX_LEN), 1)
    mask_ref[...] = (p_iota < 18 + lenb).astype(jnp.int32)


def _tc_assemble(t, base, lens2):
    return pl.pallas_call(
        _tc_body,
        grid=(N_CLS // BC,),
        in_specs=[
            pl.BlockSpec((BC, TAIL, D), lambda i: (i, 0, 0)),
            pl.BlockSpec((HEAD, D), lambda i: (0, 0)),
            pl.BlockSpec((BC, 1), lambda i: (i, 0)),
        ],
        out_specs=[
            pl.BlockSpec((BC, MAX_LEN, D), lambda i: (i, 0, 0)),
            pl.BlockSpec((BC, MAX_LEN), lambda i: (i, 0)),
        ],
        out_shape=[
            jax.ShapeDtypeStruct((N_CLS, MAX_LEN, D), jnp.float32),
            jax.ShapeDtypeStruct((N_CLS, MAX_LEN), jnp.int32),
        ],
    )(t, base, lens2)


def kernel(table, ctx, class_tokens, lens, cls_id, sep_id):
    par = (jnp.zeros((8,), jnp.int32)
           .at[0].set(jnp.asarray(cls_id, jnp.int32))
           .at[1].set(jnp.asarray(sep_id, jnp.int32)))
    t = _sc_gather(table, class_tokens, lens, par)
    t = t.reshape(N_CLS, TAIL, D)
    base = jnp.concatenate([table[cls_id][None, :], ctx], axis=0)
    out_embeds, out_mask = _tc_assemble(t, base, lens[:, None])
    return out_embeds, out_mask


# speedup vs baseline: 1.1116x; 1.0075x over previous
import functools

import jax
import jax.numpy as jnp
from jax import lax
from jax.experimental import pallas as pl
from jax.experimental.pallas import tpu as pltpu
from jax.experimental.pallas import tpu_sc as plsc

N_CLS = 4096
N_CTX = 16
MAX_NAME = 16
D = 768
MAX_LEN = 1 + N_CTX + MAX_NAME + 1
HEAD = 1 + N_CTX
TAIL = MAX_NAME + 1

NC = 2
NS = 16
NW = NC * NS
HALF = N_CLS // 2
PER_TILE = HALF // NW
K = 2
NBUF = 4
STEPS = PER_TILE // (K * NBUF)


def _sc_body(table_hbm, ct_hbm, lens_hbm, par_hbm,
             t_hbm,
             stag0, stag1, stag2, stag3,
             gidx0, gidx1, gidx2, gidx3,
             ct_v, lens_v, par_v,
             gsem0, gsem1, gsem2, gsem3,
             osem0, osem1, osem2, osem3):
    stags = (stag0, stag1, stag2, stag3)
    gidxs = (gidx0, gidx1, gidx2, gidx3)
    gsems = (gsem0, gsem1, gsem2, gsem3)
    osems = (osem0, osem1, osem2, osem3)

    wid = lax.axis_index("s") * NC + lax.axis_index("c")
    base = wid * PER_TILE
    iota = lax.broadcasted_iota(jnp.int32, (16,), 0)

    pltpu.sync_copy(par_hbm, par_v)
    pltpu.sync_copy(ct_hbm.at[pl.ds(base, PER_TILE)], ct_v)
    pltpu.sync_copy(lens_hbm.at[pl.ds(base, PER_TILE)], lens_v)
    sep_v = plsc.load_gather(par_v, [iota * 0 + 1])

    def fill_idx(b, g):
        for c in range(K):
            local = g * K + c
            lsp = jnp.full((16,), local, jnp.int32)
            tok = plsc.load_gather(ct_v, [lsp, iota])
            lenv = plsc.load_gather(lens_v, [lsp])
            idx16 = jnp.where(iota < lenv, tok, sep_v)
            plsc.store_scatter(gidxs[b], [iota * 0 + (c * TAIL) + iota], idx16)
            plsc.store_scatter(gidxs[b], [iota * 0 + (c * TAIL + 16)], sep_v,
                               mask=iota == 0)

    def step(s, carry):
        for b in range(NBUF):
            g = s * NBUF + b

            @pl.when(g >= NBUF)
            def _():
                pltpu.make_async_copy(
                    stags[b], t_hbm.at[pl.ds(0, K * TAIL)], osems[b]).wait()

            fill_idx(b, g)
            pltpu.async_copy(table_hbm.at[gidxs[b]], stags[b], gsems[b])
        for b in range(NBUF):
            g = s * NBUF + b
            r0 = (base + g * K) * TAIL
            pltpu.make_async_copy(
                table_hbm.at[gidxs[b]], stags[b], gsems[b]).wait()
            pltpu.async_copy(stags[b], t_hbm.at[pl.ds(r0, K * TAIL)], osems[b])
        return carry

    lax.fori_loop(0, STEPS, step, 0)
    for b in range(NBUF):
        pltpu.make_async_copy(stags[b], t_hbm.at[pl.ds(0, K * TAIL)],
                              osems[b]).wait()


def _sc_gather(table, class_tokens, lens, par):
    mesh = plsc.VectorSubcoreMesh(core_axis_name="c", subcore_axis_name="s")
    f = pl.kernel(
        _sc_body,
        mesh=mesh,
        compiler_params=pltpu.CompilerParams(use_tc_tiling_on_sc=False,
                                             needs_layout_passes=False),
        out_type=jax.ShapeDtypeStruct((HALF * TAIL, D), jnp.float32),
        scratch_types=(
            [pltpu.VMEM((K * TAIL, D), jnp.float32)] * NBUF
            + [pltpu.VMEM((K * TAIL,), jnp.int32)] * NBUF
            + [
                pltpu.VMEM((PER_TILE, MAX_NAME), jnp.int32),
                pltpu.VMEM((PER_TILE,), jnp.int32),
                pltpu.VMEM((8,), jnp.int32),
            ]
            + [pltpu.SemaphoreType.DMA] * (2 * NBUF)
        ),
    )
    return f(table, class_tokens, lens, par)


BC = 64


def _tc_compute(t_ref, base_ref, lens_ref, out_ref, mask_ref):
    lenb = lens_ref[...]
    s_iota = lax.broadcasted_iota(jnp.int32, (BC, TAIL, 1), 1)
    tail = jnp.where(s_iota <= lenb[:, :, None], t_ref[...], 0.0)
    head = jnp.broadcast_to(base_ref[...][None], (BC, HEAD, D))
    out_ref[...] = jnp.concatenate([head, tail], axis=1)
    p_iota = lax.broadcasted_iota(jnp.int32, (BC, MAX_LEN), 1)
    mask_ref[...] = (p_iota < 18 + lenb).astype(jnp.int32)


_OUT_SHAPES = [
    jax.ShapeDtypeStruct((N_CLS, MAX_LEN, D), jnp.float32),
    jax.ShapeDtypeStruct((N_CLS, MAX_LEN), jnp.int32),
]
_OUT_SPECS = [
    pl.BlockSpec((BC, MAX_LEN, D), lambda i: (i, 0, 0)),
    pl.BlockSpec((BC, MAX_LEN), lambda i: (i, 0)),
]
_OUT_SPECS_HI = [
    pl.BlockSpec((BC, MAX_LEN, D), lambda i: (i + HALF // BC, 0, 0)),
    pl.BlockSpec((BC, MAX_LEN), lambda i: (i + HALF // BC, 0)),
]
_IN_SPECS = [
    pl.BlockSpec((BC, TAIL, D), lambda i: (i, 0, 0)),
    pl.BlockSpec((HEAD, D), lambda i: (0, 0)),
    pl.BlockSpec((BC, 1), lambda i: (i, 0)),
]


def _tc_assemble_lo(t, base, lens2):
    return pl.pallas_call(
        _tc_compute,
        grid=(HALF // BC,),
        in_specs=_IN_SPECS,
        out_specs=_OUT_SPECS,
        out_shape=_OUT_SHAPES,
    )(t, base, lens2)


def _tc_body_hi(t_ref, base_ref, lens_ref, _prev_out, _prev_mask,
                out_ref, mask_ref):
    _tc_compute(t_ref, base_ref, lens_ref, out_ref, mask_ref)


def _tc_assemble_hi(t, base, lens2, prev_out, prev_mask):
    return pl.pallas_call(
        _tc_body_hi,
        grid=(HALF // BC,),
        in_specs=_IN_SPECS + [
            pl.BlockSpec(memory_space=pltpu.MemorySpace.HBM),
            pl.BlockSpec(memory_space=pltpu.MemorySpace.HBM),
        ],
        out_specs=_OUT_SPECS_HI,
        out_shape=_OUT_SHAPES,
        input_output_aliases={3: 0, 4: 1},
    )(t, base, lens2, prev_out, prev_mask)


def kernel(table, ctx, class_tokens, lens, cls_id, sep_id):
    par = (jnp.zeros((8,), jnp.int32)
           .at[0].set(jnp.asarray(cls_id, jnp.int32))
           .at[1].set(jnp.asarray(sep_id, jnp.int32)))
    base = jnp.concatenate([table[cls_id][None, :], ctx], axis=0)
    t0 = _sc_gather(table, class_tokens[:HALF], lens[:HALF], par)
    t1 = _sc_gather(table, class_tokens[HALF:], lens[HALF:], par)
    out_a, mask_a = _tc_assemble_lo(
        t0.reshape(HALF, TAIL, D), base, lens[:HALF, None])
    out_embeds, out_mask = _tc_assemble_hi(
        t1.reshape(HALF, TAIL, D), base, lens[HALF:, None], out_a, mask_a)
    return out_embeds, out_mask
